# baseline (speedup 1.0000x reference)
import functools

import jax
import jax.numpy as jnp
from jax import lax
from jax.experimental import pallas as pl
from jax.experimental.pallas import tpu as pltpu
from jax.experimental.pallas import tpu_sc as plsc

_NN = 13


def kernel(num_features, cat_features, tables):
    N = num_features.shape[0]
    F, V, D = tables.shape
    d_out = _NN + F * D

    tab = tables.reshape(F * V, D)
    idx = cat_features.astype(jnp.int32) + jnp.arange(F, dtype=jnp.int32) * V
    idxg = idx.reshape(N * F // 128, 128)
    num_pad = jnp.pad(num_features, ((0, 0), (0, 16 - _NN)))

    NW = 32
    RW = N // NW
    RC = 64
    NCH = RW // RC
    GB = RC * F // 128

    mesh = plsc.VectorSubcoreMesh(core_axis_name="c", subcore_axis_name="s")

    @functools.partial(
        pl.kernel,
        out_type=jax.ShapeDtypeStruct((N, d_out), jnp.float32),
        mesh=mesh,
        scratch_types=[
            pltpu.VMEM((GB, 128), jnp.int32),
            pltpu.VMEM((RC * F, D), jnp.float32),
            pltpu.VMEM((RC, 16), jnp.float32),
            pltpu.VMEM((RC, d_out), jnp.float32),
            pltpu.SemaphoreType.DMA,
        ],
        compiler_params=pltpu.CompilerParams(use_tc_tiling_on_sc=False),
    )
    def _embed(tab_hbm, idxg_hbm, num_hbm, out_hbm,
               idx_v, emb_v, num_v, out_c, sem):
        wid = lax.axis_index("s") * 2 + lax.axis_index("c")
        w_r0 = wid * RW

        def chunk_body(c, _):
            r0 = w_r0 + c * RC
            pltpu.sync_copy(idxg_hbm.at[pl.ds(r0 * F // 128, GB)], idx_v)
            pltpu.sync_copy(num_hbm.at[pl.ds(r0, RC), :], num_v)
            copies = [
                pltpu.async_copy(
                    tab_hbm.at[idx_v.at[j]],
                    emb_v.at[pl.ds(j * 128, 128), :],
                    sem,
                )
                for j in range(GB)
            ]
            for cp in copies:
                cp.wait()

            def row_body(r, _):
                out_c[r, pl.ds(0, 16)] = num_v[r, :]
                for k in range(F):
                    out_c[r, pl.ds(_NN + k * D, 16)] = emb_v[r * F + k, :]
                return 0

            lax.fori_loop(0, RC, row_body, 0, unroll=False)
            pltpu.sync_copy(out_c, out_hbm.at[pl.ds(r0, RC), :])
            return 0

        lax.fori_loop(0, NCH, chunk_body, 0, unroll=False)

    return _embed(tab, idxg, num_pad)


# speedup vs baseline: 2.1903x; 2.1903x over previous
import functools

import jax
import jax.numpy as jnp
from jax import lax
from jax.experimental import pallas as pl
from jax.experimental.pallas import tpu as pltpu
from jax.experimental.pallas import tpu_sc as plsc

_NN = 13


def kernel(num_features, cat_features, tables):
    N = num_features.shape[0]
    F, V, D = tables.shape
    d_out = _NN + F * D

    tab_t = jnp.transpose(tables, (0, 2, 1)).reshape(F * D, V)
    cat_t = jnp.transpose(cat_features, (1, 0)).astype(jnp.int32)
    num_t = jnp.transpose(num_features, (1, 0))

    NW = 32
    SW = F * D // NW
    NC = 2048
    NCH = N // NC

    mesh = plsc.VectorSubcoreMesh(core_axis_name="c", subcore_axis_name="s")

    @functools.partial(
        pl.kernel,
        out_type=jax.ShapeDtypeStruct((d_out, N), jnp.float32),
        mesh=mesh,
        scratch_types=[
            pltpu.VMEM((V,), jnp.float32),
            pltpu.VMEM((NC,), jnp.int32),
            pltpu.VMEM((NC,), jnp.float32),
            pltpu.SemaphoreType.DMA,
        ],
        compiler_params=pltpu.CompilerParams(
            use_tc_tiling_on_sc=False, needs_layout_passes=False
        ),
    )
    def _embed(tab_hbm, cat_hbm, num_hbm, out_hbm, slice_v, cat_v, out_v, sem):
        wid = lax.axis_index("s") * 2 + lax.axis_index("c")

        @pl.when(wid < _NN)
        def _():
            pltpu.sync_copy(num_hbm.at[wid, :], slice_v.at[pl.ds(0, N)])
            pltpu.sync_copy(slice_v.at[pl.ds(0, N)], out_hbm.at[wid, :])

        def slice_body(i, _):
            s = wid * SW + i
            f = s // D
            pltpu.sync_copy(tab_hbm.at[s, :], slice_v)

            def chunk_body(c, _):
                n0 = c * NC
                pltpu.sync_copy(cat_hbm.at[f, pl.ds(n0, NC)], cat_v)

                def vec_body(i16, _):
                    o = i16 * 16
                    out_v[pl.ds(o, 16)] = plsc.load_gather(
                        slice_v, [cat_v[pl.ds(o, 16)]]
                    )
                    return 0

                lax.fori_loop(0, NC // 16, vec_body, 0, unroll=4)
                pltpu.sync_copy(out_v, out_hbm.at[_NN + s, pl.ds(n0, NC)])
                return 0

            lax.fori_loop(0, NCH, chunk_body, 0, unroll=False)
            return 0

        lax.fori_loop(0, SW, slice_body, 0, unroll=False)

    out_t = _embed(tab_t, cat_t, num_t)
    return jnp.transpose(out_t, (1, 0))


# speedup vs baseline: 2.5801x; 1.1779x over previous
import functools

import jax
import jax.numpy as jnp
from jax import lax
from jax.experimental import pallas as pl
from jax.experimental.pallas import tpu as pltpu
from jax.experimental.pallas import tpu_sc as plsc

_NN = 13


def kernel(num_features, cat_features, tables):
    N = num_features.shape[0]
    F, V, D = tables.shape
    d_out = _NN + F * D

    tab_t = jnp.transpose(tables, (0, 2, 1)).reshape(F * D, V)
    cat_t = jnp.transpose(cat_features, (1, 0)).astype(jnp.int32)
    num_t = jnp.transpose(num_features, (1, 0))

    NW = 32
    SW = F * D // NW
    NC = 2048
    NCH = N // NC

    mesh = plsc.VectorSubcoreMesh(core_axis_name="c", subcore_axis_name="s")

    @functools.partial(
        pl.kernel,
        out_type=jax.ShapeDtypeStruct((d_out, N), jnp.float32),
        mesh=mesh,
        scratch_types=[
            pltpu.VMEM((V,), jnp.float32),
            pltpu.VMEM((N,), jnp.int32),
            pltpu.VMEM((4, NC), jnp.float32),
            pltpu.SemaphoreType.DMA,
            pltpu.SemaphoreType.DMA,
        ],
        compiler_params=pltpu.CompilerParams(
            use_tc_tiling_on_sc=False, needs_layout_passes=False
        ),
    )
    def _embed(tab_hbm, cat_hbm, num_hbm, out_hbm,
               slice_v, cat_v, out_v, sem, osem):
        wid = lax.axis_index("s") * 2 + lax.axis_index("c")

        @pl.when(wid < _NN)
        def _():
            pltpu.sync_copy(num_hbm.at[wid, :], slice_v.at[pl.ds(0, N)])
            pltpu.sync_copy(slice_v.at[pl.ds(0, N)], out_hbm.at[wid, :])

        def slice_body(i, f_loaded):
            s = wid * SW + i
            f = s // D

            @pl.when(f != f_loaded)
            def _():
                pltpu.sync_copy(cat_hbm.at[f, :], cat_v)

            pltpu.sync_copy(tab_hbm.at[s, :], slice_v)

            for c in range(NCH):
                n0 = c * NC
                b = c % 4
                if c >= 4:
                    pltpu.make_async_copy(
                        out_v.at[b], out_hbm.at[0, pl.ds(0, NC)], osem
                    ).wait()

                def vec_body(i16, _):
                    o = i16 * 16
                    out_v[b, pl.ds(o, 16)] = plsc.load_gather(
                        slice_v, [cat_v[pl.ds(n0 + o, 16)]]
                    )
                    return 0

                lax.fori_loop(0, NC // 16, vec_body, 0, unroll=8)
                pltpu.async_copy(
                    out_v.at[b], out_hbm.at[_NN + s, pl.ds(n0, NC)], osem
                )
            for c in range(NCH - 4, NCH):
                b = c % 4
                pltpu.make_async_copy(
                    out_v.at[b], out_hbm.at[0, pl.ds(0, NC)], osem
                ).wait()
            return f

        lax.fori_loop(0, SW, slice_body, jnp.int32(-1), unroll=False)

    out_t = _embed(tab_t, cat_t, num_t)
    return jnp.transpose(out_t, (1, 0))
